# baseline (speedup 1.0000x reference)
import functools

import jax
import jax.numpy as jnp
from jax import lax
from jax.experimental import pallas as pl
from jax.experimental.pallas import tpu as pltpu
from jax.experimental.pallas import tpu_sc as plsc

FEATURES = 1024
BATCH = 4
SEQ_LEN = 4096


@functools.lru_cache(maxsize=None)
def _make_gather(B, D):
    info = plsc.get_sparse_core_info()
    NC, NS = info.num_cores, info.num_subcores
    NW = NC * NS
    b_per_w = B // NW
    C = 32
    n_chunks = b_per_w // C
    NBUF = 2
    mesh = plsc.VectorSubcoreMesh(core_axis_name="c", subcore_axis_name="s")

    @functools.partial(
        pl.kernel,
        mesh=mesh,
        out_type=jax.ShapeDtypeStruct((B, D), jnp.float32),
        scratch_types=[
            pltpu.VMEM((b_per_w,), jnp.int32),
            [pltpu.VMEM((C, D), jnp.float32) for _ in range(NBUF)],
            pltpu.SemaphoreType.DMA,
            pltpu.SemaphoreType.DMA,
        ],
    )
    def gather_kernel(idx_hbm, table_hbm, out_hbm, idx_v, rows, gsem, wsem):
        wid = lax.axis_index("s") * NC + lax.axis_index("c")
        base = wid * b_per_w
        pltpu.sync_copy(idx_hbm.at[pl.ds(base, b_per_w)], idx_v)

        def gather(c):
            return pltpu.async_copy(
                table_hbm.at[idx_v.at[pl.ds(c * C, C)]], rows[c % NBUF], gsem
            )

        g = gather(0)
        w = [None] * n_chunks
        for c in range(n_chunks):
            g.wait()
            if c + 1 < n_chunks:
                if c >= NBUF - 1:
                    w[c - (NBUF - 1)].wait()
                g = gather(c + 1)
            w[c] = pltpu.async_copy(
                rows[c % NBUF], out_hbm.at[pl.ds(base + c * C, C)], wsem
            )
        for c in range(n_chunks - NBUF + 1, n_chunks):
            w[c].wait()

    return gather_kernel


def kernel(inputs, embedding):
    B = inputs.shape[0] * inputs.shape[1]
    D = embedding.shape[1]
    flat_idx = inputs.reshape(B)
    out = _make_gather(B, D)(flat_idx, embedding)
    return out.reshape(inputs.shape[0], inputs.shape[1], D)


# speedup vs baseline: 1.0474x; 1.0474x over previous
import functools

import jax
import jax.numpy as jnp
from jax import lax
from jax.experimental import pallas as pl
from jax.experimental.pallas import tpu as pltpu
from jax.experimental.pallas import tpu_sc as plsc

FEATURES = 1024
BATCH = 4
SEQ_LEN = 4096


@functools.lru_cache(maxsize=None)
def _make_gather(B, D):
    info = plsc.get_sparse_core_info()
    NC, NS = info.num_cores, info.num_subcores
    NW = NC * NS
    b_per_w = B // NW
    C = 16
    n_chunks = b_per_w // C
    NBUF = 4
    G = 2
    mesh = plsc.VectorSubcoreMesh(core_axis_name="c", subcore_axis_name="s")

    @functools.partial(
        pl.kernel,
        mesh=mesh,
        out_type=jax.ShapeDtypeStruct((B, D), jnp.float32),
        scratch_types=[
            pltpu.VMEM((b_per_w,), jnp.int32),
            [pltpu.VMEM((C, D), jnp.float32) for _ in range(NBUF)],
            pltpu.SemaphoreType.DMA,
            pltpu.SemaphoreType.DMA,
        ],
    )
    def gather_kernel(idx_hbm, table_hbm, out_hbm, idx_v, rows, gsem, wsem):
        wid = lax.axis_index("s") * NC + lax.axis_index("c")
        base = wid * b_per_w
        pltpu.sync_copy(idx_hbm.at[pl.ds(base, b_per_w)], idx_v)

        def gather(c):
            return pltpu.async_copy(
                table_hbm.at[idx_v.at[pl.ds(c * C, C)]], rows[c % NBUF], gsem
            )

        g = [None] * n_chunks
        w = [None] * n_chunks
        w_drained = [False] * n_chunks
        for c in range(min(G, n_chunks)):
            g[c] = gather(c)
        for c in range(n_chunks):
            g[c].wait()
            w[c] = pltpu.async_copy(
                rows[c % NBUF], out_hbm.at[pl.ds(base + c * C, C)], wsem
            )
            nxt = c + G
            if nxt < n_chunks:
                prev = nxt - NBUF
                if prev >= 0:
                    w[prev].wait()
                    w_drained[prev] = True
                g[nxt] = gather(nxt)
        for c in range(n_chunks):
            if not w_drained[c]:
                w[c].wait()

    return gather_kernel


def kernel(inputs, embedding):
    B = inputs.shape[0] * inputs.shape[1]
    D = embedding.shape[1]
    flat_idx = inputs.reshape(B)
    out = _make_gather(B, D)(flat_idx, embedding)
    return out.reshape(inputs.shape[0], inputs.shape[1], D)


# speedup vs baseline: 1.0619x; 1.0139x over previous
import functools

import jax
import jax.numpy as jnp
from jax import lax
from jax.experimental import pallas as pl
from jax.experimental.pallas import tpu as pltpu
from jax.experimental.pallas import tpu_sc as plsc

FEATURES = 1024
BATCH = 4
SEQ_LEN = 4096


@functools.lru_cache(maxsize=None)
def _make_gather(B, D):
    info = plsc.get_sparse_core_info()
    NC, NS = info.num_cores, info.num_subcores
    NW = NC * NS
    b_per_w = B // NW
    C = 16
    n_chunks = b_per_w // C
    NBA = 3
    NBS = 4
    mesh = plsc.VectorSubcoreMesh(core_axis_name="c", subcore_axis_name="s")

    @functools.partial(
        pl.kernel,
        mesh=mesh,
        out_type=jax.ShapeDtypeStruct((B, D), jnp.float32),
        scratch_types=[
            pltpu.VMEM((b_per_w,), jnp.int32),
            [pltpu.VMEM((C, D), jnp.float32) for _ in range(NBA)],
            pltpu.VMEM_SHARED((NS, NBS, C, D), jnp.float32),
            pltpu.SemaphoreType.DMA,
            pltpu.SemaphoreType.DMA,
        ],
    )
    def gather_kernel(idx_hbm, table_hbm, out_hbm, idx_v, rows, slab,
                      gsem, wsem):
        cid = lax.axis_index("c")
        sid = lax.axis_index("s")
        wid = sid * NC + cid
        base = wid * b_per_w
        pltpu.sync_copy(idx_hbm.at[pl.ds(base, b_per_w)], idx_v)

        def gather(c):
            return pltpu.async_copy(
                table_hbm.at[idx_v.at[pl.ds(c * C, C)]], rows[c % NBA], gsem
            )

        g = [None] * n_chunks
        w = [None] * n_chunks
        w_drained = [False] * n_chunks
        for c in range(min(NBA, n_chunks)):
            g[c] = gather(c)
        for c in range(n_chunks):
            g[c].wait()
            if c >= NBS:
                w[c - NBS].wait()
                w_drained[c - NBS] = True
            pltpu.sync_copy(rows[c % NBA], slab.at[sid, c % NBS])
            if c + NBA < n_chunks:
                g[c + NBA] = gather(c + NBA)
            w[c] = pltpu.async_copy(
                slab.at[sid, c % NBS], out_hbm.at[pl.ds(base + c * C, C)],
                wsem,
            )
        for c in range(n_chunks):
            if not w_drained[c]:
                w[c].wait()

    return gather_kernel


def kernel(inputs, embedding):
    B = inputs.shape[0] * inputs.shape[1]
    D = embedding.shape[1]
    flat_idx = inputs.reshape(B)
    out = _make_gather(B, D)(flat_idx, embedding)
    return out.reshape(inputs.shape[0], inputs.shape[1], D)
